# baseline (speedup 1.0000x reference)
import functools
import jax
import jax.numpy as jnp
from jax import lax
from jax.experimental import pallas as pl
from jax.experimental.pallas import tpu as pltpu, tpu_sc as plsc

_CH = 32
_NBUF = 3


def kernel(x, W):
    B, S, H = x.shape
    info = plsc.get_sparse_core_info()
    NW = info.num_cores * info.num_subcores
    rows_per_w = S // NW
    n = rows_per_w // _CH
    mesh = plsc.VectorSubcoreMesh(core_axis_name="c", subcore_axis_name="s")

    @functools.partial(
        pl.kernel, mesh=mesh,
        out_type=jax.ShapeDtypeStruct((B, S, H), W.dtype),
        scratch_types=[
            pltpu.VMEM((_NBUF, _CH, H), W.dtype),
            pltpu.SemaphoreType.DMA((_NBUF,)),
            pltpu.SemaphoreType.DMA((_NBUF,)),
        ],
    )
    def body(w_hbm, out_hbm, buf, in_sems, out_sems):
        wid = lax.axis_index("s") * info.num_cores + lax.axis_index("c")
        base = wid * rows_per_w

        def start_in(c, slot):
            cp = pltpu.make_async_copy(
                w_hbm.at[pl.ds(base + c * _CH, _CH)], buf.at[slot],
                in_sems.at[slot])
            cp.start()
            return cp

        def start_writes(c, slot):
            cps = [
                pltpu.make_async_copy(
                    buf.at[slot],
                    out_hbm.at[b, pl.ds(base + c * _CH, _CH)],
                    out_sems.at[slot])
                for b in range(B)
            ]
            for w in cps:
                w.start()
            return cps

        pending = [None] * _NBUF
        reads = [None] * _NBUF
        reads[0] = start_in(0, 0)
        for c in range(n):
            slot = c % _NBUF
            if c + 1 < n:
                nslot = (c + 1) % _NBUF
                if pending[nslot]:
                    for w in pending[nslot]:
                        w.wait()
                    pending[nslot] = None
                reads[nslot] = start_in(c + 1, nslot)
            reads[slot].wait()
            pending[slot] = start_writes(c, slot)
        for cps in pending:
            if cps:
                for w in cps:
                    w.wait()

    return body(W[:S])
